# Initial kernel scaffold; baseline (speedup 1.0000x reference)
import functools

import jax
import jax.numpy as jnp
from jax import lax
from jax.experimental import pallas as pl
from jax.experimental.pallas import tpu as pltpu
from jax.experimental.pallas import tpu_sc as plsc

N = 10000
E = 320000
D = 128
H = 128
C = 40

NC = 2
NS = 16
NW = NC * NS
EPW = E // NW
K = 125
NCH = EPW // K
NPAD = 10240
STRIPE = NPAD // NS
ZR = 64
DW = 16


def _zero_rows(ref, rows, cols):
    def body(i, carry):
        for j in range(cols // 16):
            ref[i, pl.ds(j * 16, 16)] = jnp.zeros((16,), jnp.float32)
        return carry
    lax.fori_loop(0, rows, body, 0)


def _deg_body(dst_hbm, out_hbm, dstv, onesv, zbuf, acc_sh):
    c = lax.axis_index("c")
    s = lax.axis_index("s")
    wid = c * NS + s

    def fill_ones(i, carry):
        onesv[i, pl.ds(0, 16)] = jnp.ones((16,), jnp.float32)
        return carry
    lax.fori_loop(0, K, fill_ones, 0)
    _zero_rows(zbuf, ZR, DW)
    base = s * STRIPE
    for t in range(STRIPE // ZR):
        pltpu.sync_copy(zbuf, acc_sh.at[pl.ds(base + t * ZR, ZR)])
    pltpu.sync_copy(dst_hbm.at[wid], dstv)
    plsc.subcore_barrier()

    def chunk(j, carry):
        pltpu.sync_copy(onesv, acc_sh.at[dstv.at[j]], add=True)
        return carry
    lax.fori_loop(0, NCH, chunk, 0)
    plsc.subcore_barrier()
    pltpu.sync_copy(acc_sh.at[pl.ds(base, STRIPE)],
                    out_hbm.at[pl.ds(c * NPAD + base, STRIPE)])


def _seg_body(zs_hbm, src_hbm, dst_hbm, out_hbm,
              srcv, dstv, rows0, rows1, zbuf, acc_sh, sem0, sem1):
    c = lax.axis_index("c")
    s = lax.axis_index("s")
    wid = c * NS + s

    _zero_rows(zbuf, ZR, H)
    base = s * STRIPE
    for t in range(STRIPE // ZR):
        pltpu.sync_copy(zbuf, acc_sh.at[pl.ds(base + t * ZR, ZR)])
    pltpu.sync_copy(src_hbm.at[wid], srcv)
    pltpu.sync_copy(dst_hbm.at[wid], dstv)
    plsc.subcore_barrier()

    pltpu.async_copy(zs_hbm.at[srcv.at[0]], rows0, sem0)
    pltpu.async_copy(zs_hbm.at[srcv.at[1]], rows1, sem1)

    def chunk(jj, carry):
        j0 = jj * 2
        j1 = j0 + 1
        pltpu.make_async_copy(zs_hbm.at[srcv.at[j0]], rows0, sem0).wait()
        pltpu.sync_copy(rows0, acc_sh.at[dstv.at[j0]], add=True)

        @pl.when(jj < NCH // 2 - 1)
        def _():
            pltpu.async_copy(zs_hbm.at[srcv.at[j0 + 2]], rows0, sem0)

        pltpu.make_async_copy(zs_hbm.at[srcv.at[j1]], rows1, sem1).wait()
        pltpu.sync_copy(rows1, acc_sh.at[dstv.at[j1]], add=True)

        @pl.when(jj < NCH // 2 - 1)
        def _():
            pltpu.async_copy(zs_hbm.at[srcv.at[j1 + 2]], rows1, sem1)
        return carry
    lax.fori_loop(0, NCH // 2, chunk, 0)

    plsc.subcore_barrier()
    pltpu.sync_copy(acc_sh.at[pl.ds(base, STRIPE)],
                    out_hbm.at[pl.ds(c * NPAD + base, STRIPE)])


def _sc_mesh():
    return plsc.VectorSubcoreMesh(core_axis_name="c", subcore_axis_name="s",
                                  num_cores=NC, num_subcores=NS)


def _deg_call(dst3):
    fn = pl.kernel(
        _deg_body,
        out_type=jax.ShapeDtypeStruct((NC * NPAD, DW), jnp.float32),
        mesh=_sc_mesh(),
        scratch_types=[
            pltpu.VMEM((NCH, K), jnp.int32),
            pltpu.VMEM((K, DW), jnp.float32),
            pltpu.VMEM((ZR, DW), jnp.float32),
            pltpu.VMEM_SHARED((NPAD, DW), jnp.float32),
        ],
        name="sc_degree",
    )
    return fn(dst3)


def _seg_call(zs, src3, dst3):
    fn = pl.kernel(
        _seg_body,
        out_type=jax.ShapeDtypeStruct((NC * NPAD, H), jnp.float32),
        mesh=_sc_mesh(),
        scratch_types=[
            pltpu.VMEM((NCH, K), jnp.int32),
            pltpu.VMEM((NCH, K), jnp.int32),
            pltpu.VMEM((K, H), jnp.float32),
            pltpu.VMEM((K, H), jnp.float32),
            pltpu.VMEM((ZR, H), jnp.float32),
            pltpu.VMEM_SHARED((NPAD, H), jnp.float32),
            pltpu.SemaphoreType.DMA,
            pltpu.SemaphoreType.DMA,
        ],
        name="sc_segsum",
    )
    return fn(zs, src3, dst3)


B = 2000
GRID = N // B
_MM = (((1,), (1,)), ((), ()))


def _row_spec(cols):
    return pl.BlockSpec((B, cols), lambda i: (i, 0))


def _full_spec(r, cols):
    return pl.BlockSpec((r, cols), lambda i: (0, 0))


def _a_body(dega, degb, x, w, dis_o, zs_o):
    deg = dega[...][:, 0:1] + degb[...][:, 0:1] + 1.0
    dis = lax.rsqrt(deg)
    z = lax.dot_general(x[...], w[...], _MM, preferred_element_type=jnp.float32)
    dis_o[...] = jnp.broadcast_to(dis, (B, H))
    zs_o[...] = dis * z


def _b_body(ua, ub, zs, dis, b, w, h_o, zs1_o):
    u = ua[...] + ub[...] + zs[...]
    h = jnp.maximum(dis[...] * u + b[...], 0.0)
    h_o[...] = h
    z1 = lax.dot_general(h, w[...], _MM, preferred_element_type=jnp.float32)
    zs1_o[...] = dis[...] * z1


def _c_body(ua, ub, zs, dis, b, h0, w, zs2_o):
    u = ua[...] + ub[...] + zs[...]
    h1 = jnp.maximum(dis[...] * u + b[...], 0.0) + h0[...]
    z2 = lax.dot_general(h1, w[...], _MM, preferred_element_type=jnp.float32)
    zs2_o[...] = dis[...] * z2


def _d_body(ua, ub, zs, dis, b, wm, bm, out_o):
    u = ua[...] + ub[...] + zs[...]
    h2 = dis[...] * u + b[...]
    out_o[...] = (
        lax.dot_general(h2, wm[...], _MM, preferred_element_type=jnp.float32)
        + bm[...]
    )


def _stage_a(dega, degb, x, w0):
    return pl.pallas_call(
        _a_body,
        grid=(GRID,),
        in_specs=[_row_spec(DW), _row_spec(DW), _row_spec(D), _full_spec(H, D)],
        out_specs=[_row_spec(H), _row_spec(H)],
        out_shape=[jax.ShapeDtypeStruct((N, H), jnp.float32),
                   jax.ShapeDtypeStruct((N, H), jnp.float32)],
    )(dega, degb, x, w0)


def _stage_b(ua, ub, zs, dis, b0, w1):
    return pl.pallas_call(
        _b_body,
        grid=(GRID,),
        in_specs=[_row_spec(H), _row_spec(H), _row_spec(H), _row_spec(H),
                  _full_spec(1, H), _full_spec(H, H)],
        out_specs=[_row_spec(H), _row_spec(H)],
        out_shape=[jax.ShapeDtypeStruct((N, H), jnp.float32),
                   jax.ShapeDtypeStruct((N, H), jnp.float32)],
    )(ua, ub, zs, dis, b0, w1)


def _stage_c(ua, ub, zs, dis, b1, h0, w2):
    return pl.pallas_call(
        _c_body,
        grid=(GRID,),
        in_specs=[_row_spec(H), _row_spec(H), _row_spec(H), _row_spec(H),
                  _full_spec(1, H), _row_spec(H), _full_spec(H, H)],
        out_specs=_row_spec(H),
        out_shape=jax.ShapeDtypeStruct((N, H), jnp.float32),
    )(ua, ub, zs, dis, b1, h0, w2)


def _stage_d(ua, ub, zs, dis, b2, wm, bm):
    return pl.pallas_call(
        _d_body,
        grid=(GRID,),
        in_specs=[_row_spec(H), _row_spec(H), _row_spec(H), _row_spec(H),
                  _full_spec(1, H), _full_spec(C, H), _full_spec(1, C)],
        out_specs=pl.BlockSpec((B, C), lambda i: (i, 0)),
        out_shape=jax.ShapeDtypeStruct((N, C), jnp.float32),
    )(ua, ub, zs, dis, b2, wm, bm)


def kernel(X, A, W0, b0, W1, b1, W2, b2, Wm, bm):
    src3 = A[0].reshape(NW, NCH, K)
    dst3 = A[1].reshape(NW, NCH, K)

    degs = _deg_call(dst3)
    dis_b, zs0 = _stage_a(degs[:NPAD], degs[NPAD:], X, W0)

    u0 = _seg_call(zs0, src3, dst3)
    h0, zs1 = _stage_b(u0[:N], u0[NPAD:NPAD + N], zs0, dis_b,
                       b0.reshape(1, H), W1)

    u1 = _seg_call(zs1, src3, dst3)
    zs2 = _stage_c(u1[:N], u1[NPAD:NPAD + N], zs1, dis_b,
                   b1.reshape(1, H), h0, W2)

    u2 = _seg_call(zs2, src3, dst3)
    return _stage_d(u2[:N], u2[NPAD:NPAD + N], zs2, dis_b,
                    b2.reshape(1, H), Wm, bm.reshape(1, C))


# speedup vs baseline: 21.8964x; 21.8964x over previous
import jax
import jax.numpy as jnp
from jax import lax
from jax.experimental import pallas as pl
from jax.experimental.pallas import tpu as pltpu
from jax.experimental.pallas import tpu_sc as plsc

N = 10000
E = 320000
D = 128
H = 128
C = 40

NC = 2
NS = 16
NW = NC * NS
FW = H // NC
NPAD = 16000
STRIPE = NPAD // NS
ZR = 50
DW = 16

EPT = E // NS
KS = 125
NCHS = EPT // KS

EPW = E // NW
KD = 125
NCHD = EPW // KD


def _zero_rows(ref, rows, cols):
    def body(i, carry):
        for j in range(cols // 16):
            ref[i, pl.ds(j * 16, 16)] = jnp.zeros((16,), jnp.float32)
        return carry
    lax.fori_loop(0, rows, body, 0)


def _deg_body(dst_hbm, out_hbm, dstv, onesv, zbuf, acc_sh):
    c = lax.axis_index("c")
    s = lax.axis_index("s")
    wid = c * NS + s

    def fill_ones(i, carry):
        onesv[i, pl.ds(0, 16)] = jnp.ones((16,), jnp.float32)
        return carry
    lax.fori_loop(0, KD, fill_ones, 0)
    _zero_rows(zbuf, ZR, DW)
    base = s * STRIPE
    for t in range(STRIPE // ZR):
        pltpu.sync_copy(zbuf, acc_sh.at[pl.ds(base + t * ZR, ZR)])
    pltpu.sync_copy(dst_hbm.at[wid], dstv)
    plsc.subcore_barrier()

    def chunk(j, carry):
        pltpu.sync_copy(onesv, acc_sh.at[dstv.at[j]], add=True)
        return carry
    lax.fori_loop(0, NCHD, chunk, 0)
    plsc.subcore_barrier()
    pltpu.sync_copy(acc_sh.at[pl.ds(base, STRIPE)],
                    out_hbm.at[pl.ds(c * NPAD + base, STRIPE)])


def _seg_body(zs_hbm, src_hbm, dst_hbm, out_hbm,
              srcv, dstv, rows0, rows1, zbuf, acc_sh, sem0, sem1):
    c = lax.axis_index("c")
    s = lax.axis_index("s")

    _zero_rows(zbuf, ZR, FW)
    base = s * STRIPE
    for t in range(STRIPE // ZR):
        pltpu.sync_copy(zbuf, acc_sh.at[pl.ds(base + t * ZR, ZR)])
    pltpu.sync_copy(src_hbm.at[c * NS + s], srcv)
    pltpu.sync_copy(dst_hbm.at[s], dstv)
    plsc.subcore_barrier()

    pltpu.async_copy(zs_hbm.at[srcv.at[0]], rows0, sem0)
    pltpu.async_copy(zs_hbm.at[srcv.at[1]], rows1, sem1)

    def chunk(jj, carry):
        j0 = jj * 2
        j1 = j0 + 1
        pltpu.make_async_copy(zs_hbm.at[srcv.at[j0]], rows0, sem0).wait()
        pltpu.sync_copy(rows0, acc_sh.at[dstv.at[j0]], add=True)

        @pl.when(jj < NCHS // 2 - 1)
        def _():
            pltpu.async_copy(zs_hbm.at[srcv.at[j0 + 2]], rows0, sem0)

        pltpu.make_async_copy(zs_hbm.at[srcv.at[j1]], rows1, sem1).wait()
        pltpu.sync_copy(rows1, acc_sh.at[dstv.at[j1]], add=True)

        @pl.when(jj < NCHS // 2 - 1)
        def _():
            pltpu.async_copy(zs_hbm.at[srcv.at[j1 + 2]], rows1, sem1)
        return carry
    lax.fori_loop(0, NCHS // 2, chunk, 0)

    plsc.subcore_barrier()
    pltpu.sync_copy(acc_sh.at[pl.ds(base, STRIPE)],
                    out_hbm.at[pl.ds(c * NPAD + base, STRIPE)])


def _sc_mesh():
    return plsc.VectorSubcoreMesh(core_axis_name="c", subcore_axis_name="s",
                                  num_cores=NC, num_subcores=NS)


def _deg_call(dst3):
    fn = pl.kernel(
        _deg_body,
        out_type=jax.ShapeDtypeStruct((NC * NPAD, DW), jnp.float32),
        mesh=_sc_mesh(),
        scratch_types=[
            pltpu.VMEM((NCHD, KD), jnp.int32),
            pltpu.VMEM((KD, DW), jnp.float32),
            pltpu.VMEM((ZR, DW), jnp.float32),
            pltpu.VMEM_SHARED((NPAD, DW), jnp.float32),
        ],
        compiler_params=pltpu.CompilerParams(use_tc_tiling_on_sc=False),
        name="sc_degree",
    )
    return fn(dst3)


def _seg_call(zs2n, src4, dst3):
    fn = pl.kernel(
        _seg_body,
        out_type=jax.ShapeDtypeStruct((NC * NPAD, FW), jnp.float32),
        mesh=_sc_mesh(),
        scratch_types=[
            pltpu.VMEM((NCHS, KS), jnp.int32),
            pltpu.VMEM((NCHS, KS), jnp.int32),
            pltpu.VMEM((KS, FW), jnp.float32),
            pltpu.VMEM((KS, FW), jnp.float32),
            pltpu.VMEM((ZR, FW), jnp.float32),
            pltpu.VMEM_SHARED((NPAD, FW), jnp.float32),
            pltpu.SemaphoreType.DMA,
            pltpu.SemaphoreType.DMA,
        ],
        compiler_params=pltpu.CompilerParams(use_tc_tiling_on_sc=False),
        name="sc_segsum",
    )
    return fn(zs2n, src4, dst3)


B = 2000
GRID = N // B
UOFF = NPAD // B
_MM = (((1,), (1,)), ((), ()))


def _row_spec(cols):
    return pl.BlockSpec((B, cols), lambda i: (i, 0))


def _u_spec(which):
    return pl.BlockSpec((B, FW), lambda i, w=which: (i + w * UOFF, 0))


def _half_spec():
    return pl.BlockSpec((2, B, FW), lambda i: (0, i, 0))


def _full_spec(r, cols):
    return pl.BlockSpec((r, cols), lambda i: (0, 0))


def _split(x):
    return jnp.stack([x[:, :FW], x[:, FW:]], axis=0)


def _merge(ul, ur, zs3):
    z = zs3[...]
    return jnp.concatenate([ul[...] + z[0], ur[...] + z[1]], axis=1)


def _a_body(dega, degb, x, w, dis_o, zs_o):
    deg = dega[...][:, 0:1] + degb[...][:, 0:1] + 1.0
    dis = lax.rsqrt(deg)
    z = lax.dot_general(x[...], w[...], _MM, preferred_element_type=jnp.float32)
    dis_o[...] = jnp.broadcast_to(dis, (B, H))
    zs_o[...] = _split(dis * z)


def _b_body(ul, ur, zs3, dis, b, w, h_o, zs1_o):
    u = _merge(ul, ur, zs3)
    h = jnp.maximum(dis[...] * u + b[...], 0.0)
    h_o[...] = h
    z1 = lax.dot_general(h, w[...], _MM, preferred_element_type=jnp.float32)
    zs1_o[...] = _split(dis[...] * z1)


def _c_body(ul, ur, zs3, dis, b, h0, w, zs2_o):
    u = _merge(ul, ur, zs3)
    h1 = jnp.maximum(dis[...] * u + b[...], 0.0) + h0[...]
    z2 = lax.dot_general(h1, w[...], _MM, preferred_element_type=jnp.float32)
    zs2_o[...] = _split(dis[...] * z2)


def _d_body(ul, ur, zs3, dis, b, wm, bm, out_o):
    u = _merge(ul, ur, zs3)
    h2 = dis[...] * u + b[...]
    out_o[...] = (
        lax.dot_general(h2, wm[...], _MM, preferred_element_type=jnp.float32)
        + bm[...]
    )


def _stage_b(u, zs3, dis, b0, w1):
    return pl.pallas_call(
        _b_body,
        grid=(GRID,),
        in_specs=[_u_spec(0), _u_spec(1), _half_spec(), _row_spec(H),
                  _full_spec(1, H), _full_spec(H, H)],
        out_specs=[_row_spec(H), _half_spec()],
        out_shape=[jax.ShapeDtypeStruct((N, H), jnp.float32),
                   jax.ShapeDtypeStruct((2, N, FW), jnp.float32)],
    )(u, u, zs3, dis, b0, w1)


def _stage_c(u, zs3, dis, b1, h0, w2):
    return pl.pallas_call(
        _c_body,
        grid=(GRID,),
        in_specs=[_u_spec(0), _u_spec(1), _half_spec(), _row_spec(H),
                  _full_spec(1, H), _row_spec(H), _full_spec(H, H)],
        out_specs=_half_spec(),
        out_shape=jax.ShapeDtypeStruct((2, N, FW), jnp.float32),
    )(u, u, zs3, dis, b1, h0, w2)


def _stage_d(u, zs3, dis, b2, wm, bm):
    return pl.pallas_call(
        _d_body,
        grid=(GRID,),
        in_specs=[_u_spec(0), _u_spec(1), _half_spec(), _row_spec(H),
                  _full_spec(1, H), _full_spec(C, H), _full_spec(1, C)],
        out_specs=pl.BlockSpec((B, C), lambda i: (i, 0)),
        out_shape=jax.ShapeDtypeStruct((N, C), jnp.float32),
    )(u, u, zs3, dis, b2, wm, bm)


def _deg_spec(which):
    return pl.BlockSpec((B, DW), lambda i, w=which: (i + w * UOFF, 0))


def kernel(X, A, W0, b0, W1, b1, W2, b2, Wm, bm):
    src, dst = A[0], A[1]
    src_t = src.reshape(NS, NCHS, KS)
    src4 = jnp.concatenate([src_t, src_t + N], axis=0)
    dst3s = dst.reshape(NS, NCHS, KS)
    dst3d = dst.reshape(NW, NCHD, KD)

    degs = _deg_call(dst3d)
    dis_b, zs0_3 = pl.pallas_call(
        _a_body,
        grid=(GRID,),
        in_specs=[_deg_spec(0), _deg_spec(1), _row_spec(D), _full_spec(H, D)],
        out_specs=[_row_spec(H), _half_spec()],
        out_shape=[jax.ShapeDtypeStruct((N, H), jnp.float32),
                   jax.ShapeDtypeStruct((2, N, FW), jnp.float32)],
    )(degs, degs, X, W0)

    u0 = _seg_call(zs0_3.reshape(2 * N, FW), src4, dst3s)
    h0, zs1_3 = _stage_b(u0, zs0_3, dis_b, b0.reshape(1, H), W1)

    u1 = _seg_call(zs1_3.reshape(2 * N, FW), src4, dst3s)
    zs2_3 = _stage_c(u1, zs1_3, dis_b, b1.reshape(1, H), h0, W2)

    u2 = _seg_call(zs2_3.reshape(2 * N, FW), src4, dst3s)
    return _stage_d(u2, zs2_3, dis_b, b2.reshape(1, H), Wm, bm.reshape(1, C))


# speedup vs baseline: 25.7756x; 1.1772x over previous
import jax
import jax.numpy as jnp
from jax import lax
from jax.experimental import pallas as pl
from jax.experimental.pallas import tpu as pltpu
from jax.experimental.pallas import tpu_sc as plsc

N = 10000
E = 320000
D = 128
H = 128
C = 40

NC = 2
NS = 16
NW = NC * NS
FW = H // NC
NPAD = 16000
STRIPE = NPAD // NS
ZR = 50
DW = 16

EPT = E // NS
KS = 125
NCHS = EPT // KS

EPW = E // NW
KD = 125
NCHD = EPW // KD


def _zero_rows(ref, rows, cols):
    def body(i, carry):
        for j in range(cols // 16):
            ref[i, pl.ds(j * 16, 16)] = jnp.zeros((16,), jnp.float32)
        return carry
    lax.fori_loop(0, rows, body, 0)


def _deg_body(dst_hbm, out_hbm, dstv, onesv, zbuf, acc_sh):
    c = lax.axis_index("c")
    s = lax.axis_index("s")
    wid = c * NS + s

    def fill_ones(i, carry):
        onesv[i, pl.ds(0, 16)] = jnp.ones((16,), jnp.float32)
        return carry
    lax.fori_loop(0, KD, fill_ones, 0)
    _zero_rows(zbuf, ZR, DW)
    base = s * STRIPE
    for t in range(STRIPE // ZR):
        pltpu.sync_copy(zbuf, acc_sh.at[pl.ds(base + t * ZR, ZR)])
    pltpu.sync_copy(dst_hbm.at[wid], dstv)
    plsc.subcore_barrier()

    def chunk(j, carry):
        pltpu.sync_copy(onesv, acc_sh.at[dstv.at[j]], add=True)
        return carry
    lax.fori_loop(0, NCHD, chunk, 0)
    plsc.subcore_barrier()
    pltpu.sync_copy(acc_sh.at[pl.ds(base, STRIPE)],
                    out_hbm.at[pl.ds(base, STRIPE), pl.ds(c * FW, DW)])


def _seg_body(zs_hbm, src_hbm, dst_hbm, out_hbm,
              srcv, dstv, rows0, rows1, zbuf, acc_sh, sem0, sem1):
    c = lax.axis_index("c")
    s = lax.axis_index("s")

    _zero_rows(zbuf, ZR, FW)
    base = s * STRIPE
    for t in range(STRIPE // ZR):
        pltpu.sync_copy(zbuf, acc_sh.at[pl.ds(base + t * ZR, ZR)])
    pltpu.sync_copy(src_hbm.at[c * NS + s], srcv)
    pltpu.sync_copy(dst_hbm.at[s], dstv)
    plsc.subcore_barrier()

    pltpu.async_copy(zs_hbm.at[srcv.at[0]], rows0, sem0)
    pltpu.async_copy(zs_hbm.at[srcv.at[1]], rows1, sem1)

    def chunk(jj, carry):
        j0 = jj * 2
        j1 = j0 + 1
        pltpu.make_async_copy(zs_hbm.at[srcv.at[j0]], rows0, sem0).wait()
        pltpu.sync_copy(rows0, acc_sh.at[dstv.at[j0]], add=True)

        @pl.when(jj < NCHS // 2 - 1)
        def _():
            pltpu.async_copy(zs_hbm.at[srcv.at[j0 + 2]], rows0, sem0)

        pltpu.make_async_copy(zs_hbm.at[srcv.at[j1]], rows1, sem1).wait()
        pltpu.sync_copy(rows1, acc_sh.at[dstv.at[j1]], add=True)

        @pl.when(jj < NCHS // 2 - 1)
        def _():
            pltpu.async_copy(zs_hbm.at[srcv.at[j1 + 2]], rows1, sem1)
        return carry
    lax.fori_loop(0, NCHS // 2, chunk, 0)

    plsc.subcore_barrier()
    pltpu.sync_copy(acc_sh.at[pl.ds(base, STRIPE)],
                    out_hbm.at[pl.ds(base, STRIPE), pl.ds(c * FW, FW)])


def _sc_mesh():
    return plsc.VectorSubcoreMesh(core_axis_name="c", subcore_axis_name="s",
                                  num_cores=NC, num_subcores=NS)


def _deg_call(dst3):
    fn = pl.kernel(
        _deg_body,
        out_type=jax.ShapeDtypeStruct((NPAD, H), jnp.float32),
        mesh=_sc_mesh(),
        scratch_types=[
            pltpu.VMEM((NCHD, KD), jnp.int32),
            pltpu.VMEM((KD, DW), jnp.float32),
            pltpu.VMEM((ZR, DW), jnp.float32),
            pltpu.VMEM_SHARED((NPAD, DW), jnp.float32),
        ],
        compiler_params=pltpu.CompilerParams(use_tc_tiling_on_sc=False),
        name="sc_degree",
    )
    return fn(dst3)


def _seg_call(zs2n, src4, dst3):
    fn = pl.kernel(
        _seg_body,
        out_type=jax.ShapeDtypeStruct((NPAD, H), jnp.float32),
        mesh=_sc_mesh(),
        scratch_types=[
            pltpu.VMEM((NCHS, KS), jnp.int32),
            pltpu.VMEM((NCHS, KS), jnp.int32),
            pltpu.VMEM((KS, FW), jnp.float32),
            pltpu.VMEM((KS, FW), jnp.float32),
            pltpu.VMEM((ZR, FW), jnp.float32),
            pltpu.VMEM_SHARED((NPAD, FW), jnp.float32),
            pltpu.SemaphoreType.DMA,
            pltpu.SemaphoreType.DMA,
        ],
        compiler_params=pltpu.CompilerParams(use_tc_tiling_on_sc=False),
        name="sc_segsum",
    )
    return fn(zs2n, src4, dst3)


B = 2000
GRID = N // B
_MM = (((1,), (1,)), ((), ()))


def _row_spec(cols):
    return pl.BlockSpec((B, cols), lambda i: (i, 0))


def _full_spec(r, cols):
    return pl.BlockSpec((r, cols), lambda i: (0, 0))


def _a_body(degp, x, w, dis_o, zs_o):
    d = degp[...]
    deg = d[:, 0:1] + d[:, FW:FW + 1] + 1.0
    dis = lax.rsqrt(deg)
    z = lax.dot_general(x[...], w[...], _MM, preferred_element_type=jnp.float32)
    dis_o[...] = jnp.broadcast_to(dis, (B, H))
    zs_o[...] = dis * z


def _b_body(u, zs, dis, b, w, h_o, zs1_o):
    h = jnp.maximum(dis[...] * (u[...] + zs[...]) + b[...], 0.0)
    h_o[...] = h
    z1 = lax.dot_general(h, w[...], _MM, preferred_element_type=jnp.float32)
    zs1_o[...] = dis[...] * z1


def _c_body(u, zs, dis, b, h0, w, zs2_o):
    h1 = jnp.maximum(dis[...] * (u[...] + zs[...]) + b[...], 0.0) + h0[...]
    z2 = lax.dot_general(h1, w[...], _MM, preferred_element_type=jnp.float32)
    zs2_o[...] = dis[...] * z2


def _d_body(u, zs, dis, b, wm, bm, out_o):
    h2 = dis[...] * (u[...] + zs[...]) + b[...]
    out_o[...] = (
        lax.dot_general(h2, wm[...], _MM, preferred_element_type=jnp.float32)
        + bm[...]
    )


def _stage_a(degp, x, w0):
    return pl.pallas_call(
        _a_body,
        grid=(GRID,),
        in_specs=[_row_spec(H), _row_spec(D), _full_spec(H, D)],
        out_specs=[_row_spec(H), _row_spec(H)],
        out_shape=[jax.ShapeDtypeStruct((N, H), jnp.float32),
                   jax.ShapeDtypeStruct((N, H), jnp.float32)],
    )(degp, x, w0)


def _stage_b(u, zs, dis, b0, w1):
    return pl.pallas_call(
        _b_body,
        grid=(GRID,),
        in_specs=[_row_spec(H), _row_spec(H), _row_spec(H),
                  _full_spec(1, H), _full_spec(H, H)],
        out_specs=[_row_spec(H), _row_spec(H)],
        out_shape=[jax.ShapeDtypeStruct((N, H), jnp.float32),
                   jax.ShapeDtypeStruct((N, H), jnp.float32)],
    )(u, zs, dis, b0, w1)


def _stage_c(u, zs, dis, b1, h0, w2):
    return pl.pallas_call(
        _c_body,
        grid=(GRID,),
        in_specs=[_row_spec(H), _row_spec(H), _row_spec(H),
                  _full_spec(1, H), _row_spec(H), _full_spec(H, H)],
        out_specs=_row_spec(H),
        out_shape=jax.ShapeDtypeStruct((N, H), jnp.float32),
    )(u, zs, dis, b1, h0, w2)


def _stage_d(u, zs, dis, b2, wm, bm):
    return pl.pallas_call(
        _d_body,
        grid=(GRID,),
        in_specs=[_row_spec(H), _row_spec(H), _row_spec(H),
                  _full_spec(1, H), _full_spec(C, H), _full_spec(1, C)],
        out_specs=pl.BlockSpec((B, C), lambda i: (i, 0)),
        out_shape=jax.ShapeDtypeStruct((N, C), jnp.float32),
    )(u, zs, dis, b2, wm, bm)


def kernel(X, A, W0, b0, W1, b1, W2, b2, Wm, bm):
    src, dst = A[0], A[1]
    src_t = (2 * src).reshape(NS, NCHS, KS)
    src4 = jnp.concatenate([src_t, src_t + 1], axis=0)
    dst3s = dst.reshape(NS, NCHS, KS)
    dst3d = dst.reshape(NW, NCHD, KD)

    degp = _deg_call(dst3d)
    dis_b, zs0 = _stage_a(degp, X, W0)

    u0 = _seg_call(zs0.reshape(2 * N, FW), src4, dst3s)
    h0, zs1 = _stage_b(u0, zs0, dis_b, b0.reshape(1, H), W1)

    u1 = _seg_call(zs1.reshape(2 * N, FW), src4, dst3s)
    zs2 = _stage_c(u1, zs1, dis_b, b1.reshape(1, H), h0, W2)

    u2 = _seg_call(zs2.reshape(2 * N, FW), src4, dst3s)
    return _stage_d(u2, zs2, dis_b, b2.reshape(1, H), Wm, bm.reshape(1, C))
